# baseline (speedup 1.0000x reference)
import functools

import jax
import jax.numpy as jnp
from jax import lax
from jax.experimental import pallas as pl
from jax.experimental.pallas import tpu as pltpu
from jax.experimental.pallas import tpu_sc as plsc

N = 10000
E = 320000
R = 16
F = 128
EMB = 64

NC = 2
NS = 16
L = 16

EP = E // NS
K = 128
NB = 160
EPAD = NB * K
NBUF = 2

NN = N // NC
RA = NN * R
DEAD = RA
ACC_ROWS = 80128
DSTRIPE = RA // NS
ZROWS = 200
NFC = F // L
WN = 2000
PK = 1 << 17
G = 4
GK = G * K
PBUF = EPAD + 2096


_XB = 1000


def _proj_body(x_ref, w_ref, re_ref, b_ref, p1_ref, p2_ref, r3_ref):
    xb = x_ref[...]
    p1_ref[...] = jnp.dot(xb, w_ref[0:F, :], preferred_element_type=jnp.float32)
    p2_ref[...] = jnp.dot(xb, w_ref[F:2 * F, :], preferred_element_type=jnp.float32)

    @pl.when(pl.program_id(0) == 0)
    def _():
        r3_ref[...] = (
            jnp.dot(re_ref[...], w_ref[2 * F:, :], preferred_element_type=jnp.float32)
            + b_ref[...]
        )


def _proj(x, rel_emb, W, b2d):
    return pl.pallas_call(
        _proj_body,
        grid=(N // _XB,),
        in_specs=[
            pl.BlockSpec((_XB, F), lambda i: (i, 0)),
            pl.BlockSpec((2 * F + EMB, F), lambda i: (0, 0)),
            pl.BlockSpec((R, EMB), lambda i: (0, 0)),
            pl.BlockSpec((1, F), lambda i: (0, 0)),
        ],
        out_specs=[
            pl.BlockSpec((_XB, F), lambda i: (i, 0)),
            pl.BlockSpec((_XB, F), lambda i: (i, 0)),
            pl.BlockSpec((R, F), lambda i: (0, 0)),
        ],
        out_shape=[
            jax.ShapeDtypeStruct((N, F), jnp.float32),
            jax.ShapeDtypeStruct((N, F), jnp.float32),
            jax.ShapeDtypeStruct((R, F), jnp.float32),
        ],
    )(x, W, rel_emb, b2d)


def _sc_body(p1v, srch, dsth, typh, s2a, s2c,
             packedb, dstw, typw, srcw,
             ridd0, ridd1, idxd0, idxd1,
             rb0, rb1, zb, accs,
             g0, g1, s0, s1):
    c = lax.axis_index("c")
    s = lax.axis_index("s")
    base = s * EP
    lo = c * NN
    rbufs = (rb0, rb1)
    ridd = (ridd0, ridd1)
    idxd = (idxd0, idxd1)
    gsems = (g0, g1)
    ssems = (s0, s1)

    off = jnp.int32(0)
    for w in range(EP // WN):
        wb = base + w * WN
        pltpu.async_copy(dsth.at[pl.ds(wb, WN)], dstw, g0)
        pltpu.async_copy(typh.at[pl.ds(wb, WN)], typw, g1)
        pltpu.async_copy(srch.at[pl.ds(wb, WN)], srcw, s0)
        pltpu.make_async_copy(dsth.at[pl.ds(wb, WN)], dstw, g0).wait()
        pltpu.make_async_copy(typh.at[pl.ds(wb, WN)], typw, g1).wait()
        pltpu.make_async_copy(srch.at[pl.ds(wb, WN)], srcw, s0).wait()

        def cstep(i, o):
            d = dstw[pl.ds(i * L, L)]
            t = typw[pl.ds(i * L, L)]
            sv = srcw[pl.ds(i * L, L)]
            dl = d - lo
            m = (dl >= 0) & (dl < NN)
            packed = sv * PK + (dl * R + t)
            cs = plsc.cumsum(jnp.where(m, jnp.int32(1), jnp.int32(0)))
            plsc.store_scatter(packedb, [o + cs - 1], packed, mask=m)
            return o + cs[L - 1]

        off = lax.fori_loop(0, WN // L, cstep, off)

    nbat = jnp.maximum((off + (GK - 1)) // GK, 4)
    nbat4 = ((nbat + 3) // 4) * 4
    npadg = (nbat4 * GK - off + (L - 1)) // L

    def pstep(i, carry):
        packedb[pl.ds(off + i * L, L)] = jnp.full((L,), DEAD, jnp.int32)
        return carry

    lax.fori_loop(0, npadg, pstep, 0)

    def zero_fill(i, carry):
        zb[i, :] = jnp.zeros((L,), jnp.float32)
        return carry

    lax.fori_loop(0, ZROWS, zero_fill, 0)

    def _zero_stripe():
        for kk in range(DSTRIPE // ZROWS):
            pltpu.sync_copy(
                zb, accs.at[pl.ds(s * DSTRIPE + kk * ZROWS, ZROWS)])
        pltpu.sync_copy(zb.at[pl.ds(0, 8)], accs.at[pl.ds(DEAD + s * 8, 8)])

    _zero_stripe()
    plsc.subcore_barrier()

    def _decode(j, bslot, fc, with_idx):
        def dk(i, carry):
            v = packedb[pl.ds(j * GK + i * L, L)]
            ridd[bslot][pl.ds(i * L, L)] = v & (PK - 1)
            if with_idx:
                idxd[bslot][pl.ds(i * L, L)] = (v // PK) * NFC + fc
            return carry

        lax.fori_loop(0, GK // L, dk, 0)

    def _gather(bslot):
        pltpu.async_copy(p1v.at[idxd[bslot]], rbufs[bslot], gsems[bslot])

    def _gather_wait(bslot):
        pltpu.make_async_copy(
            p1v.at[idxd[bslot]], rbufs[bslot], gsems[bslot]).wait()

    def _scatter(bslot):
        pltpu.async_copy(
            rbufs[bslot], accs.at[ridd[bslot]], ssems[bslot], add=True)

    def _scatter_wait(bslot):
        pltpu.make_async_copy(
            rbufs[bslot], accs.at[ridd[bslot]], ssems[bslot]).wait()

    for fc in range(NFC):
        _decode(0, 0, fc, True)
        _gather(0)

        def pass_body(o, carry):
            for kk in range(NBUF):
                j = o * NBUF + kk
                bp = (kk + 1) % NBUF

                @pl.when(j >= 1)
                def _():
                    _scatter_wait(bp)

                @pl.when(j + 1 < nbat4)
                def _():
                    _decode(j + 1, bp, fc, True)
                    _gather(bp)

                _gather_wait(kk)
                _scatter(kk)
            return carry

        lax.fori_loop(0, nbat4 // NBUF, pass_body, 0)
        _scatter_wait(1)
        plsc.subcore_barrier()

        pltpu.sync_copy(
            accs.at[pl.ds(s * DSTRIPE, DSTRIPE)],
            s2a.at[pl.ds(c * RA + s * DSTRIPE, DSTRIPE), pl.ds(fc * L, L)],
        )
        _zero_stripe()
        plsc.subcore_barrier()

    def ones_fill(i, carry):
        rb0[i, :] = jnp.full((L,), 1.0, jnp.float32)
        return carry

    lax.fori_loop(0, GK, ones_fill, 0)

    def _cscatter(bslot):
        pltpu.async_copy(rb0, accs.at[ridd[bslot]], ssems[bslot], add=True)

    def _cscatter_wait(bslot):
        pltpu.make_async_copy(rb0, accs.at[ridd[bslot]], ssems[bslot]).wait()

    def cnt_body(o, carry):
        for kk in range(NBUF):
            j = o * NBUF + kk

            @pl.when(j >= NBUF)
            def _():
                _cscatter_wait(kk)

            _decode(j, kk, 0, False)
            _cscatter(kk)
        return carry

    lax.fori_loop(0, nbat4 // NBUF, cnt_body, 0)
    for kk in range(NBUF):
        _cscatter_wait(kk)
    plsc.subcore_barrier()
    pltpu.sync_copy(
        accs.at[pl.ds(s * DSTRIPE, DSTRIPE)],
        s2c.at[pl.ds(c * RA + s * DSTRIPE, DSTRIPE)],
    )


_sc_call = functools.partial(
    pl.kernel,
    out_type=[
        jax.ShapeDtypeStruct((N * R, F), jnp.float32),
        jax.ShapeDtypeStruct((N * R, L), jnp.float32),
    ],
    mesh=plsc.VectorSubcoreMesh(core_axis_name="c", subcore_axis_name="s"),
    scratch_types=[
        pltpu.VMEM((PBUF,), jnp.int32),
        pltpu.VMEM((WN,), jnp.int32),
        pltpu.VMEM((WN,), jnp.int32),
        pltpu.VMEM((WN,), jnp.int32),
        pltpu.VMEM((GK,), jnp.int32),
        pltpu.VMEM((GK,), jnp.int32),
        pltpu.VMEM((GK,), jnp.int32),
        pltpu.VMEM((GK,), jnp.int32),
        pltpu.VMEM((GK, L), jnp.float32),
        pltpu.VMEM((GK, L), jnp.float32),
        pltpu.VMEM((ZROWS, L), jnp.float32),
        pltpu.VMEM_SHARED((ACC_ROWS, L), jnp.float32),
        pltpu.SemaphoreType.DMA,
        pltpu.SemaphoreType.DMA,
        pltpu.SemaphoreType.DMA,
        pltpu.SemaphoreType.DMA,
    ],
    compiler_params=pltpu.CompilerParams(
        use_tc_tiling_on_sc=False, needs_layout_passes=False),
)(_sc_body)


_FB = 400


def _fin_body(s_ref, c_ref, p2_ref, r3_ref, o_ref):
    sv = s_ref[...]
    cnt = c_ref[:, :, 0:1]
    p2 = p2_ref[...][:, None, :]
    r3 = r3_ref[...][None, :, :]
    agg = sv + cnt * (p2 + r3)
    o_ref[...] = jnp.tanh(agg).sum(axis=1)


def _finalize(s3, c3, P2, r3):
    return pl.pallas_call(
        _fin_body,
        grid=(N // _FB,),
        in_specs=[
            pl.BlockSpec((_FB, R, F), lambda i: (i, 0, 0)),
            pl.BlockSpec((_FB, R, L), lambda i: (i, 0, 0)),
            pl.BlockSpec((_FB, F), lambda i: (i, 0)),
            pl.BlockSpec((R, F), lambda i: (0, 0)),
        ],
        out_specs=pl.BlockSpec((_FB, F), lambda i: (i, 0)),
        out_shape=jax.ShapeDtypeStruct((N, F), jnp.float32),
    )(s3, c3, P2, r3)


def kernel(x, edge_index, edge_type, rel_emb, W, b):
    P1, P2, r3 = _proj(x, rel_emb, W, b.reshape(1, F))
    p1v = P1.reshape(N * NFC, L)
    src = edge_index[0].astype(jnp.int32)
    dst = edge_index[1].astype(jnp.int32)
    typ = edge_type.astype(jnp.int32)
    s2a, s2c = _sc_call(p1v, src, dst, typ)
    return _finalize(
        s2a.reshape(N, R, F), s2c.reshape(N, R, L), P2, r3
    )


# speedup vs baseline: 1.0372x; 1.0372x over previous
import functools

import jax
import jax.numpy as jnp
from jax import lax
from jax.experimental import pallas as pl
from jax.experimental.pallas import tpu as pltpu
from jax.experimental.pallas import tpu_sc as plsc

N = 10000
E = 320000
R = 16
F = 128
EMB = 64

NC = 2
NS = 16
L = 16

EP = E // NS
K = 128
NB = 160
EPAD = NB * K
NBUF = 4

NN = N // NC
RA = NN * R
DEAD = RA
ACC_ROWS = 80128
DSTRIPE = RA // NS
ZROWS = 250
NFC = F // L
WN = 2000
PK = 1 << 17
G = 2
GK = G * K
PBUF = EPAD + 1056


_XB = 1000


def _proj_body(x_ref, w_ref, re_ref, b_ref, p1_ref, p2_ref, r3_ref):
    xb = x_ref[...]
    p1_ref[...] = jnp.dot(xb, w_ref[0:F, :], preferred_element_type=jnp.float32)
    p2_ref[...] = jnp.dot(xb, w_ref[F:2 * F, :], preferred_element_type=jnp.float32)

    @pl.when(pl.program_id(0) == 0)
    def _():
        r3_ref[...] = (
            jnp.dot(re_ref[...], w_ref[2 * F:, :], preferred_element_type=jnp.float32)
            + b_ref[...]
        )


def _proj(x, rel_emb, W, b2d):
    return pl.pallas_call(
        _proj_body,
        grid=(N // _XB,),
        in_specs=[
            pl.BlockSpec((_XB, F), lambda i: (i, 0)),
            pl.BlockSpec((2 * F + EMB, F), lambda i: (0, 0)),
            pl.BlockSpec((R, EMB), lambda i: (0, 0)),
            pl.BlockSpec((1, F), lambda i: (0, 0)),
        ],
        out_specs=[
            pl.BlockSpec((_XB, F), lambda i: (i, 0)),
            pl.BlockSpec((_XB, F), lambda i: (i, 0)),
            pl.BlockSpec((R, F), lambda i: (0, 0)),
        ],
        out_shape=[
            jax.ShapeDtypeStruct((N, F), jnp.float32),
            jax.ShapeDtypeStruct((N, F), jnp.float32),
            jax.ShapeDtypeStruct((R, F), jnp.float32),
        ],
    )(x, W, rel_emb, b2d)


def _sc_body(p1v, srch, dsth, typh, s2a, s2c,
             packedb, dstw, typw, srcw,
             ridd0, ridd1, ridd2, ridd3, idxd0, idxd1, idxd2, idxd3,
             rb0, rb1, rb2, rb3, zb, accs,
             g0, g1, g2, g3, s0, s1, s2, s3):
    c = lax.axis_index("c")
    s = lax.axis_index("s")
    base = s * EP
    lo = c * NN
    rbufs = (rb0, rb1, rb2, rb3)
    ridd = (ridd0, ridd1, ridd2, ridd3)
    idxd = (idxd0, idxd1, idxd2, idxd3)
    gsems = (g0, g1, g2, g3)
    ssems = (s0, s1, s2, s3)

    off = jnp.int32(0)
    for w in range(EP // WN):
        wb = base + w * WN
        pltpu.async_copy(dsth.at[pl.ds(wb, WN)], dstw, g0)
        pltpu.async_copy(typh.at[pl.ds(wb, WN)], typw, g1)
        pltpu.async_copy(srch.at[pl.ds(wb, WN)], srcw, g2)
        pltpu.make_async_copy(dsth.at[pl.ds(wb, WN)], dstw, g0).wait()
        pltpu.make_async_copy(typh.at[pl.ds(wb, WN)], typw, g1).wait()
        pltpu.make_async_copy(srch.at[pl.ds(wb, WN)], srcw, g2).wait()

        def cstep(i, o):
            d = dstw[pl.ds(i * L, L)]
            t = typw[pl.ds(i * L, L)]
            sv = srcw[pl.ds(i * L, L)]
            dl = d - lo
            m = (dl >= 0) & (dl < NN)
            packed = sv * PK + (dl * R + t)
            cs = plsc.cumsum(jnp.where(m, jnp.int32(1), jnp.int32(0)))
            plsc.store_scatter(packedb, [o + cs - 1], packed, mask=m)
            return o + cs[L - 1]

        off = lax.fori_loop(0, WN // L, cstep, off)

    nbat = jnp.maximum((off + (GK - 1)) // GK, 4)
    nbat4 = ((nbat + 3) // 4) * 4
    npadg = (nbat4 * GK - off + (L - 1)) // L

    def pstep(i, carry):
        packedb[pl.ds(off + i * L, L)] = jnp.full((L,), DEAD, jnp.int32)
        return carry

    lax.fori_loop(0, npadg, pstep, 0)

    def zero_fill(i, carry):
        zb[i, :] = jnp.zeros((L,), jnp.float32)
        return carry

    lax.fori_loop(0, ZROWS, zero_fill, 0)

    def _zero_stripe():
        for kk in range(DSTRIPE // ZROWS):
            pltpu.sync_copy(
                zb, accs.at[pl.ds(s * DSTRIPE + kk * ZROWS, ZROWS)])
        pltpu.sync_copy(zb.at[pl.ds(0, 8)], accs.at[pl.ds(DEAD + s * 8, 8)])

    _zero_stripe()
    plsc.subcore_barrier()

    def _decode(j, bslot, fc, with_idx):
        def dk(i, carry):
            v = packedb[pl.ds(j * GK + i * L, L)]
            ridd[bslot][pl.ds(i * L, L)] = v & (PK - 1)
            if with_idx:
                idxd[bslot][pl.ds(i * L, L)] = (v // PK) * NFC + fc
            return carry

        lax.fori_loop(0, GK // L, dk, 0)

    def _gather(bslot):
        pltpu.async_copy(p1v.at[idxd[bslot]], rbufs[bslot], gsems[bslot])

    def _gather_wait(bslot):
        pltpu.make_async_copy(
            p1v.at[idxd[bslot]], rbufs[bslot], gsems[bslot]).wait()

    def _scatter(bslot):
        pltpu.async_copy(
            rbufs[bslot], accs.at[ridd[bslot]], ssems[bslot], add=True)

    def _scatter_wait(bslot):
        pltpu.make_async_copy(
            rbufs[bslot], accs.at[ridd[bslot]], ssems[bslot]).wait()

    for fc in range(NFC):
        for bslot in range(2):
            _decode(bslot, bslot, fc, True)
            _gather(bslot)

        def pass_body(o, carry):
            for kk in range(NBUF):
                j = o * NBUF + kk
                bp = (kk + 2) % NBUF

                @pl.when(j >= 2)
                def _():
                    _scatter_wait(bp)

                @pl.when(j + 2 < nbat4)
                def _():
                    _decode(j + 2, bp, fc, True)
                    _gather(bp)

                _gather_wait(kk)
                _scatter(kk)
            return carry

        lax.fori_loop(0, nbat4 // NBUF, pass_body, 0)
        _scatter_wait(2)
        _scatter_wait(3)
        plsc.subcore_barrier()

        pltpu.sync_copy(
            accs.at[pl.ds(s * DSTRIPE, DSTRIPE)],
            s2a.at[pl.ds(c * RA + s * DSTRIPE, DSTRIPE), pl.ds(fc * L, L)],
        )
        _zero_stripe()
        plsc.subcore_barrier()

    def ones_fill(i, carry):
        rb0[i, :] = jnp.full((L,), 1.0, jnp.float32)
        return carry

    lax.fori_loop(0, GK, ones_fill, 0)

    def _cscatter(bslot):
        pltpu.async_copy(rb0, accs.at[ridd[bslot]], ssems[bslot], add=True)

    def _cscatter_wait(bslot):
        pltpu.make_async_copy(rb0, accs.at[ridd[bslot]], ssems[bslot]).wait()

    def cnt_body(o, carry):
        for kk in range(NBUF):
            j = o * NBUF + kk

            @pl.when(j >= NBUF)
            def _():
                _cscatter_wait(kk)

            _decode(j, kk, 0, False)
            _cscatter(kk)
        return carry

    lax.fori_loop(0, nbat4 // NBUF, cnt_body, 0)
    for kk in range(NBUF):
        _cscatter_wait(kk)
    plsc.subcore_barrier()
    pltpu.sync_copy(
        accs.at[pl.ds(s * DSTRIPE, DSTRIPE)],
        s2c.at[pl.ds(c * RA + s * DSTRIPE, DSTRIPE)],
    )


_sc_call = functools.partial(
    pl.kernel,
    out_type=[
        jax.ShapeDtypeStruct((N * R, F), jnp.float32),
        jax.ShapeDtypeStruct((N * R, L), jnp.float32),
    ],
    mesh=plsc.VectorSubcoreMesh(core_axis_name="c", subcore_axis_name="s"),
    scratch_types=[
        pltpu.VMEM((PBUF,), jnp.int32),
        pltpu.VMEM((WN,), jnp.int32),
        pltpu.VMEM((WN,), jnp.int32),
        pltpu.VMEM((WN,), jnp.int32),
        pltpu.VMEM((GK,), jnp.int32),
        pltpu.VMEM((GK,), jnp.int32),
        pltpu.VMEM((GK,), jnp.int32),
        pltpu.VMEM((GK,), jnp.int32),
        pltpu.VMEM((GK,), jnp.int32),
        pltpu.VMEM((GK,), jnp.int32),
        pltpu.VMEM((GK,), jnp.int32),
        pltpu.VMEM((GK,), jnp.int32),
        pltpu.VMEM((GK, L), jnp.float32),
        pltpu.VMEM((GK, L), jnp.float32),
        pltpu.VMEM((GK, L), jnp.float32),
        pltpu.VMEM((GK, L), jnp.float32),
        pltpu.VMEM((ZROWS, L), jnp.float32),
        pltpu.VMEM_SHARED((ACC_ROWS, L), jnp.float32),
        pltpu.SemaphoreType.DMA,
        pltpu.SemaphoreType.DMA,
        pltpu.SemaphoreType.DMA,
        pltpu.SemaphoreType.DMA,
        pltpu.SemaphoreType.DMA,
        pltpu.SemaphoreType.DMA,
        pltpu.SemaphoreType.DMA,
        pltpu.SemaphoreType.DMA,
    ],
    compiler_params=pltpu.CompilerParams(
        use_tc_tiling_on_sc=False, needs_layout_passes=False),
)(_sc_body)


_FB = 400


def _fin_body(s_ref, c_ref, p2_ref, r3_ref, o_ref):
    sv = s_ref[...]
    cnt = c_ref[:, :, 0:1]
    p2 = p2_ref[...][:, None, :]
    r3 = r3_ref[...][None, :, :]
    agg = sv + cnt * (p2 + r3)
    o_ref[...] = jnp.tanh(agg).sum(axis=1)


def _finalize(s3, c3, P2, r3):
    return pl.pallas_call(
        _fin_body,
        grid=(N // _FB,),
        in_specs=[
            pl.BlockSpec((_FB, R, F), lambda i: (i, 0, 0)),
            pl.BlockSpec((_FB, R, L), lambda i: (i, 0, 0)),
            pl.BlockSpec((_FB, F), lambda i: (i, 0)),
            pl.BlockSpec((R, F), lambda i: (0, 0)),
        ],
        out_specs=pl.BlockSpec((_FB, F), lambda i: (i, 0)),
        out_shape=jax.ShapeDtypeStruct((N, F), jnp.float32),
    )(s3, c3, P2, r3)


def kernel(x, edge_index, edge_type, rel_emb, W, b):
    P1, P2, r3 = _proj(x, rel_emb, W, b.reshape(1, F))
    p1v = P1.reshape(N * NFC, L)
    src = edge_index[0].astype(jnp.int32)
    dst = edge_index[1].astype(jnp.int32)
    typ = edge_type.astype(jnp.int32)
    s2a, s2c = _sc_call(p1v, src, dst, typ)
    return _finalize(
        s2a.reshape(N, R, F), s2c.reshape(N, R, L), P2, r3
    )


# speedup vs baseline: 1.2726x; 1.2269x over previous
import functools

import jax
import jax.numpy as jnp
from jax import lax
from jax.experimental import pallas as pl
from jax.experimental.pallas import tpu as pltpu
from jax.experimental.pallas import tpu_sc as plsc

N = 10000
E = 320000
R = 16
F = 128
EMB = 64

NC = 2
NS = 16
L = 16

EP = E // NS
K = 128
NB = 160
EPAD = NB * K
NBUF = 4

NN = N // NC
NCH = 2
NNC = NN // NCH
RA = NNC * R
DEAD = RA
ACC_ROWS = 40128
DSTRIPE = RA // NS
ZROWS = 100
FW = 2 * L
NFC = F // FW
WN = 2000
PK = 1 << 17
GK = 128
PBUF = EPAD + 1056


_XB = 1000


def _proj_body(x_ref, w_ref, re_ref, b_ref, p1_ref, p2_ref, r3_ref):
    xb = x_ref[...]
    p1_ref[...] = jnp.dot(xb, w_ref[0:F, :], preferred_element_type=jnp.float32)
    p2_ref[...] = jnp.dot(xb, w_ref[F:2 * F, :], preferred_element_type=jnp.float32)

    @pl.when(pl.program_id(0) == 0)
    def _():
        r3_ref[...] = (
            jnp.dot(re_ref[...], w_ref[2 * F:, :], preferred_element_type=jnp.float32)
            + b_ref[...]
        )


def _proj(x, rel_emb, W, b2d):
    return pl.pallas_call(
        _proj_body,
        grid=(N // _XB,),
        in_specs=[
            pl.BlockSpec((_XB, F), lambda i: (i, 0)),
            pl.BlockSpec((2 * F + EMB, F), lambda i: (0, 0)),
            pl.BlockSpec((R, EMB), lambda i: (0, 0)),
            pl.BlockSpec((1, F), lambda i: (0, 0)),
        ],
        out_specs=[
            pl.BlockSpec((_XB, F), lambda i: (i, 0)),
            pl.BlockSpec((_XB, F), lambda i: (i, 0)),
            pl.BlockSpec((R, F), lambda i: (0, 0)),
        ],
        out_shape=[
            jax.ShapeDtypeStruct((N, F), jnp.float32),
            jax.ShapeDtypeStruct((N, F), jnp.float32),
            jax.ShapeDtypeStruct((R, F), jnp.float32),
        ],
    )(x, W, rel_emb, b2d)


def _sc_body(p1v, srch, dsth, typh, s2a, s2c,
             packedb, dstw, typw, srcw,
             ridd0, ridd1, ridd2, ridd3, idxd0, idxd1, idxd2, idxd3,
             rb0, rb1, rb2, rb3, zb, accs,
             g0, g1, g2, g3, s0, s1, s2, s3):
    c = lax.axis_index("c")
    s = lax.axis_index("s")
    base = s * EP
    lo = c * NN
    rbufs = (rb0, rb1, rb2, rb3)
    ridd = (ridd0, ridd1, ridd2, ridd3)
    idxd = (idxd0, idxd1, idxd2, idxd3)
    gsems = (g0, g1, g2, g3)
    ssems = (s0, s1, s2, s3)

    def zero_fill(i, carry):
        zb[i, pl.ds(0, L)] = jnp.zeros((L,), jnp.float32)
        zb[i, pl.ds(L, L)] = jnp.zeros((L,), jnp.float32)
        return carry

    lax.fori_loop(0, ZROWS, zero_fill, 0)

    def _zero_stripe():
        for kk in range(DSTRIPE // ZROWS):
            pltpu.sync_copy(
                zb, accs.at[pl.ds(s * DSTRIPE + kk * ZROWS, ZROWS)])
        pltpu.sync_copy(zb.at[pl.ds(0, 8)], accs.at[pl.ds(DEAD + s * 8, 8)])

    _zero_stripe()
    plsc.subcore_barrier()

    def _decode(j, bslot, fc, with_idx):
        def dk(i, carry):
            v = packedb[pl.ds(j * GK + i * L, L)]
            ridd[bslot][pl.ds(i * L, L)] = v & (PK - 1)
            if with_idx:
                idxd[bslot][pl.ds(i * L, L)] = (v // PK) * NFC + fc
            return carry

        lax.fori_loop(0, GK // L, dk, 0)

    def _gather(bslot):
        pltpu.async_copy(p1v.at[idxd[bslot]], rbufs[bslot], gsems[bslot])

    def _gather_wait(bslot):
        pltpu.make_async_copy(
            p1v.at[idxd[bslot]], rbufs[bslot], gsems[bslot]).wait()

    def _scatter(bslot):
        pltpu.async_copy(
            rbufs[bslot], accs.at[ridd[bslot]], ssems[bslot], add=True)

    def _scatter_wait(bslot):
        pltpu.make_async_copy(
            rbufs[bslot], accs.at[ridd[bslot]], ssems[bslot]).wait()

    def _cscatter(bslot):
        pltpu.async_copy(rb0, accs.at[ridd[bslot]], ssems[bslot], add=True)

    def _cscatter_wait(bslot):
        pltpu.make_async_copy(rb0, accs.at[ridd[bslot]], ssems[bslot]).wait()

    for ch in range(NCH):
        lo = c * NN + ch * NNC
        rowbase = c * NN * R + ch * RA

        off = jnp.int32(0)
        for w in range(EP // WN):
            wb = base + w * WN
            pltpu.async_copy(dsth.at[pl.ds(wb, WN)], dstw, g0)
            pltpu.async_copy(typh.at[pl.ds(wb, WN)], typw, g1)
            pltpu.async_copy(srch.at[pl.ds(wb, WN)], srcw, g2)
            pltpu.make_async_copy(dsth.at[pl.ds(wb, WN)], dstw, g0).wait()
            pltpu.make_async_copy(typh.at[pl.ds(wb, WN)], typw, g1).wait()
            pltpu.make_async_copy(srch.at[pl.ds(wb, WN)], srcw, g2).wait()

            def cstep(i, o):
                d = dstw[pl.ds(i * L, L)]
                t = typw[pl.ds(i * L, L)]
                sv = srcw[pl.ds(i * L, L)]
                dl = d - lo
                m = (dl >= 0) & (dl < NNC)
                packed = sv * PK + (dl * R + t)
                cs = plsc.cumsum(jnp.where(m, jnp.int32(1), jnp.int32(0)))
                plsc.store_scatter(packedb, [o + cs - 1], packed, mask=m)
                return o + cs[L - 1]

            off = lax.fori_loop(0, WN // L, cstep, off)

        nbat = jnp.maximum((off + (GK - 1)) // GK, 4)
        nbat4 = ((nbat + 3) // 4) * 4
        npadg = (nbat4 * GK - off + (L - 1)) // L

        def pstep(i, carry):
            packedb[pl.ds(off + i * L, L)] = jnp.full((L,), DEAD, jnp.int32)
            return carry

        lax.fori_loop(0, npadg, pstep, 0)

        for fc in range(NFC):
            for bslot in range(2):
                _decode(bslot, bslot, fc, True)
                _gather(bslot)

            def pass_body(o, carry):
                for kk in range(NBUF):
                    j = o * NBUF + kk
                    bp = (kk + 2) % NBUF

                    @pl.when(j >= 2)
                    def _():
                        _scatter_wait(bp)

                    @pl.when(j + 2 < nbat4)
                    def _():
                        _decode(j + 2, bp, fc, True)
                        _gather(bp)

                    _gather_wait(kk)
                    _scatter(kk)
                return carry

            lax.fori_loop(0, nbat4 // NBUF, pass_body, 0)
            _scatter_wait(2)
            _scatter_wait(3)
            plsc.subcore_barrier()

            pltpu.sync_copy(
                accs.at[pl.ds(s * DSTRIPE, DSTRIPE)],
                s2a.at[pl.ds(rowbase + s * DSTRIPE, DSTRIPE),
                       pl.ds(fc * FW, FW)],
            )
            _zero_stripe()
            plsc.subcore_barrier()

        def ones_fill(i, carry):
            rb0[i, pl.ds(0, L)] = jnp.full((L,), 1.0, jnp.float32)
            rb0[i, pl.ds(L, L)] = jnp.full((L,), 1.0, jnp.float32)
            return carry

        lax.fori_loop(0, GK, ones_fill, 0)

        def cnt_body(o, carry):
            for kk in range(NBUF):
                j = o * NBUF + kk

                @pl.when(j >= NBUF)
                def _():
                    _cscatter_wait(kk)

                _decode(j, kk, 0, False)
                _cscatter(kk)
            return carry

        lax.fori_loop(0, nbat4 // NBUF, cnt_body, 0)
        for kk in range(NBUF):
            _cscatter_wait(kk)
        plsc.subcore_barrier()
        pltpu.sync_copy(
            accs.at[pl.ds(s * DSTRIPE, DSTRIPE)],
            s2c.at[pl.ds(rowbase + s * DSTRIPE, DSTRIPE)],
        )
        if ch < NCH - 1:
            _zero_stripe()
        plsc.subcore_barrier()


_sc_call = functools.partial(
    pl.kernel,
    out_type=[
        jax.ShapeDtypeStruct((N * R, F), jnp.float32),
        jax.ShapeDtypeStruct((N * R, FW), jnp.float32),
    ],
    mesh=plsc.VectorSubcoreMesh(core_axis_name="c", subcore_axis_name="s"),
    scratch_types=[
        pltpu.VMEM((PBUF,), jnp.int32),
        pltpu.VMEM((WN,), jnp.int32),
        pltpu.VMEM((WN,), jnp.int32),
        pltpu.VMEM((WN,), jnp.int32),
        pltpu.VMEM((GK,), jnp.int32),
        pltpu.VMEM((GK,), jnp.int32),
        pltpu.VMEM((GK,), jnp.int32),
        pltpu.VMEM((GK,), jnp.int32),
        pltpu.VMEM((GK,), jnp.int32),
        pltpu.VMEM((GK,), jnp.int32),
        pltpu.VMEM((GK,), jnp.int32),
        pltpu.VMEM((GK,), jnp.int32),
        pltpu.VMEM((GK, FW), jnp.float32),
        pltpu.VMEM((GK, FW), jnp.float32),
        pltpu.VMEM((GK, FW), jnp.float32),
        pltpu.VMEM((GK, FW), jnp.float32),
        pltpu.VMEM((ZROWS, FW), jnp.float32),
        pltpu.VMEM_SHARED((ACC_ROWS, FW), jnp.float32),
        pltpu.SemaphoreType.DMA,
        pltpu.SemaphoreType.DMA,
        pltpu.SemaphoreType.DMA,
        pltpu.SemaphoreType.DMA,
        pltpu.SemaphoreType.DMA,
        pltpu.SemaphoreType.DMA,
        pltpu.SemaphoreType.DMA,
        pltpu.SemaphoreType.DMA,
    ],
    compiler_params=pltpu.CompilerParams(
        use_tc_tiling_on_sc=False, needs_layout_passes=False),
)(_sc_body)


_FB = 400


def _fin_body(s_ref, c_ref, p2_ref, r3_ref, o_ref):
    sv = s_ref[...]
    cnt = c_ref[:, :, 0:1]
    p2 = p2_ref[...][:, None, :]
    r3 = r3_ref[...][None, :, :]
    agg = sv + cnt * (p2 + r3)
    o_ref[...] = jnp.tanh(agg).sum(axis=1)


def _finalize(s3, c3, P2, r3):
    return pl.pallas_call(
        _fin_body,
        grid=(N // _FB,),
        in_specs=[
            pl.BlockSpec((_FB, R, F), lambda i: (i, 0, 0)),
            pl.BlockSpec((_FB, R, FW), lambda i: (i, 0, 0)),
            pl.BlockSpec((_FB, F), lambda i: (i, 0)),
            pl.BlockSpec((R, F), lambda i: (0, 0)),
        ],
        out_specs=pl.BlockSpec((_FB, F), lambda i: (i, 0)),
        out_shape=jax.ShapeDtypeStruct((N, F), jnp.float32),
    )(s3, c3, P2, r3)


def kernel(x, edge_index, edge_type, rel_emb, W, b):
    P1, P2, r3 = _proj(x, rel_emb, W, b.reshape(1, F))
    p1v = P1.reshape(N * NFC, FW)
    src = edge_index[0].astype(jnp.int32)
    dst = edge_index[1].astype(jnp.int32)
    typ = edge_type.astype(jnp.int32)
    s2a, s2c = _sc_call(p1v, src, dst, typ)
    return _finalize(
        s2a.reshape(N, R, F), s2c.reshape(N, R, FW), P2, r3
    )
